# Initial kernel scaffold; baseline (speedup 1.0000x reference)
import jax
import jax.numpy as jnp
from jax.experimental import pallas as pl


def kernel(x, edge_index, node_centrality1, edge_centrality1, batch, assignment_index_2, iso_type_2, edge_index_2, batch_2, assignment_index_3, iso_type_3, edge_index_3, batch_3, params):
    raise NotImplementedError("write your pallas kernel here")


# speedup vs baseline: 1.0115x; 1.0115x over previous
import functools

import jax
import jax.numpy as jnp
from jax.experimental import pallas as pl
from jax.experimental.pallas import tpu as pltpu

N1 = 10000
B = 64
NCLS = 10


def _pelu(z):
    return jnp.where(z > 0, z, jnp.exp(jnp.minimum(z, 0.0)) - 1.0)


def _head_body(x1_ref, x2_ref, x3_ref, wf1_ref, bf1_ref, wf2_ref, bf2_ref,
               wf3_ref, bf3_ref, o_ref):
    z = jnp.concatenate([x1_ref[...], x2_ref[...], x3_ref[...]], axis=1)
    z = _pelu(z @ wf1_ref[...] + bf1_ref[...])
    z = _pelu(z @ wf2_ref[...] + bf2_ref[...])
    z = z @ wf3_ref[...] + bf3_ref[...]
    m = jnp.max(z, axis=1, keepdims=True)
    lse = jnp.log(jnp.sum(jnp.exp(z - m), axis=1, keepdims=True)) + m
    o_ref[...] = z - lse


def _head(x1, x2, x3, p):
    return pl.pallas_call(
        _head_body,
        out_shape=jax.ShapeDtypeStruct((B, NCLS), jnp.float32),
    )(x1, x2, x3, p['Wf1'], p['bf1'][None, :], p['Wf2'], p['bf2'][None, :],
      p['Wf3'], p['bf3'][None, :])


def _qconv(x, edge_index, nc, ec, Ws, Wn, b, n):
    src = edge_index[0]
    dst = edge_index[1]
    y = x @ Wn
    m = jnp.take(y, src, axis=0)
    if ec is not None:
        m = m * ec[:, None]
    agg = jax.ops.segment_sum(m, dst, num_segments=n)
    h = x * nc[:, None] if nc is not None else x
    return agg + h @ Ws + b


def _scatter_mean(data, seg, n):
    s = jax.ops.segment_sum(data, seg, num_segments=n)
    cnt = jax.ops.segment_sum(jnp.ones((data.shape[0],), dtype=data.dtype), seg, num_segments=n)
    return s / jnp.maximum(cnt, 1.0)[:, None]


def _avg_pool(x, assign, n):
    return _scatter_mean(jnp.take(x, assign[0], axis=0), assign[1], n)


def kernel(x, edge_index, node_centrality1, edge_centrality1, batch,
           assignment_index_2, iso_type_2, edge_index_2, batch_2,
           assignment_index_3, iso_type_3, edge_index_3, batch_3, params):
    p = params
    elu = jax.nn.elu
    h = elu(_qconv(x, edge_index, node_centrality1, edge_centrality1, p['W1s'], p['W1n'], p['b1'], N1))
    h = elu(_qconv(h, edge_index, node_centrality1, edge_centrality1, p['W2s'], p['W2n'], p['b2'], N1))
    h = elu(_qconv(h, edge_index, node_centrality1, edge_centrality1, p['W3s'], p['W3n'], p['b3'], N1))
    x1 = jax.ops.segment_sum(h, batch, num_segments=B)
    h2 = _avg_pool(h, assignment_index_2, 10000)
    h2 = jnp.concatenate([h2, iso_type_2], axis=1)
    h2 = elu(_qconv(h2, edge_index_2, None, None, p['W4s'], p['W4n'], p['b4'], 10000))
    h2 = elu(_qconv(h2, edge_index_2, None, None, p['W5s'], p['W5n'], p['b5'], 10000))
    x2 = _scatter_mean(h2, batch_2, B)
    h3 = _avg_pool(h, assignment_index_3, 10000)
    h3 = jnp.concatenate([h3, iso_type_3], axis=1)
    h3 = elu(_qconv(h3, edge_index_3, None, None, p['W6s'], p['W6n'], p['b6'], 10000))
    h3 = elu(_qconv(h3, edge_index_3, None, None, p['W7s'], p['W7n'], p['b7'], 10000))
    x3 = _scatter_mean(h3, batch_3, B)
    return _head(x1, x2, x3, p)


# speedup vs baseline: 3.3473x; 3.3092x over previous
import dataclasses
import functools

import jax
import jax.numpy as jnp
from jax import lax
from jax.experimental import pallas as pl
from jax.experimental.pallas import tpu as pltpu
from jax.experimental.pallas import tpu_sc as plsc

N = 10000
H = 64
B = 64
NCLS = 10
NC_SC = 2
NS_SC = 16
NW = NC_SC * NS_SC
CH = 80
WSC = 128
ROWS_PER_TILE = N // NS_SC
NPOOL = 10016
PROWS_PER_TILE = NPOOL // NS_SC

f32 = jnp.float32
i32 = jnp.int32


def _pelu(z):
    return jnp.where(z > 0, z, jnp.exp(jnp.minimum(z, 0.0)) - 1.0)


def _sc_mesh():
    return plsc.VectorSubcoreMesh(
        core_axis_name="c", subcore_axis_name="s",
        num_cores=NC_SC, num_subcores=NS_SC)


def _sc_params():
    cp = pltpu.CompilerParams()
    if "needs_layout_passes" in pltpu.CompilerParams.__dataclass_fields__:
        cp = dataclasses.replace(cp, needs_layout_passes=False)
    return cp


def _zero_fill(buf, nrows):
    zv = jnp.zeros((16,), f32)

    @pl.loop(0, nrows)
    def _(r):
        for j in range(WSC // 16):
            buf[r, pl.ds(j * 16, 16)] = zv


def _zero_acc(zbuf, acc_sh, base, nrows):
    _zero_fill(zbuf, CH)
    full, rem = divmod(nrows, CH)
    for k in range(full):
        pltpu.sync_copy(zbuf, acc_sh.at[pl.ds(base + k * CH, CH)])
    if rem:
        pltpu.sync_copy(zbuf.at[pl.ds(0, rem)],
                        acc_sh.at[pl.ds(base + full * CH, rem)])


def _conv_body(with_ec, epw, y_hbm, src_hbm, dst_hbm, ec_hbm, out_hbm,
               sidx, didx, ecb, rows, acc_sh, sem):
    cid = lax.axis_index("c")
    sid = lax.axis_index("s")
    wid = cid * NS_SC + sid
    nchunk = epw // CH

    _zero_acc(rows, acc_sh, sid * ROWS_PER_TILE, ROWS_PER_TILE)
    plsc.subcore_barrier()

    @pl.loop(0, nchunk)
    def _(c):
        base = wid * epw + c * CH
        pltpu.sync_copy(src_hbm.at[pl.ds(base, CH)], sidx)
        pltpu.sync_copy(dst_hbm.at[pl.ds(base, CH)], didx)
        pltpu.async_copy(y_hbm.at[sidx], rows, sem).wait()
        if with_ec:
            pltpu.sync_copy(ec_hbm.at[pl.ds(base, CH)], ecb)

            @pl.loop(0, CH)
            def _(e):
                ecv = plsc.load_gather(ecb, [jnp.full((16,), e, i32)])
                for j in range(H // 16):
                    sl = pl.ds(j * 16, 16)
                    rows[e, sl] = rows[e, sl] * ecv
        pltpu.sync_copy(rows, acc_sh.at[didx], add=True)

    plsc.subcore_barrier()
    sl = pl.ds(sid * ROWS_PER_TILE, ROWS_PER_TILE)
    pltpu.sync_copy(acc_sh.at[sl], out_hbm.at[cid, sid])


def _sc_conv(y, src, dst, ec, *, with_ec):
    epw = src.shape[0] // NW
    scratch = [
        pltpu.VMEM((CH,), i32),
        pltpu.VMEM((CH,), i32),
        pltpu.VMEM((CH,), f32),
        pltpu.VMEM((CH, WSC), f32),
        pltpu.VMEM_SHARED((N, WSC), f32),
        pltpu.SemaphoreType.DMA,
    ]
    kern = pl.kernel(
        functools.partial(_conv_body, with_ec, epw),
        out_type=jax.ShapeDtypeStruct((NC_SC, NS_SC, ROWS_PER_TILE, WSC), f32),
        mesh=_sc_mesh(),
        scratch_types=scratch,
        compiler_params=_sc_params(),
    )
    return kern(y, src, dst, ec).reshape(NC_SC, N, WSC)


def _pool_body(epw, tab_hbm, src_hbm, dst_hbm, out_hbm,
               sidx, didx, rows, acc_sh, sem):
    cid = lax.axis_index("c")
    sid = lax.axis_index("s")
    wid = cid * NS_SC + sid
    nchunk = epw // CH

    _zero_acc(rows, acc_sh, sid * PROWS_PER_TILE, PROWS_PER_TILE)
    plsc.subcore_barrier()

    @pl.loop(0, nchunk)
    def _(c):
        base = wid * epw + c * CH
        pltpu.sync_copy(src_hbm.at[pl.ds(base, CH)], sidx)
        pltpu.sync_copy(dst_hbm.at[pl.ds(base, CH)], didx)
        pltpu.async_copy(tab_hbm.at[sidx], rows, sem).wait()
        pltpu.sync_copy(rows, acc_sh.at[didx], add=True)

    plsc.subcore_barrier()
    sl = pl.ds(sid * PROWS_PER_TILE, PROWS_PER_TILE)
    pltpu.sync_copy(acc_sh.at[sl], out_hbm.at[cid, sid])


def _sc_pool(tab, src, dst):
    epw = src.shape[0] // NW
    scratch = [
        pltpu.VMEM((CH,), i32),
        pltpu.VMEM((CH,), i32),
        pltpu.VMEM((CH, WSC), f32),
        pltpu.VMEM_SHARED((NPOOL, WSC), f32),
        pltpu.SemaphoreType.DMA,
    ]
    kern = pl.kernel(
        functools.partial(_pool_body, epw),
        out_type=jax.ShapeDtypeStruct((NC_SC, NS_SC, PROWS_PER_TILE, WSC), f32),
        mesh=_sc_mesh(),
        scratch_types=scratch,
        compiler_params=_sc_params(),
    )
    return kern(tab, src, dst).reshape(NC_SC, NPOOL, WSC)


_RB = 1000
_GRID = N // _RB


def _dot(a, b):
    return lax.dot_general(a, b, (((1,), (0,)), ((), ())),
                           preferred_element_type=f32,
                           precision=lax.Precision.HIGHEST)


def _dot_t(a, b):
    return lax.dot_general(a, b, (((0,), (0,)), ((), ())),
                           preferred_element_type=f32,
                           precision=lax.Precision.HIGHEST)


def _rows_spec(w):
    return pl.BlockSpec((_RB, w), lambda i: (i, 0))


def _full_spec(r, c):
    return pl.BlockSpec((r, c), lambda i: (0, 0))


def _widen(y):
    return jnp.concatenate([y, jnp.zeros((_RB, WSC - H), f32)], axis=1)


def _pre_body(x_ref, nc_ref, wn_ref, ws_ref, y_ref, s_ref):
    xb = x_ref[...]
    y_ref[...] = _widen(_dot(xb, wn_ref[...]))
    s_ref[...] = _dot(xb * nc_ref[...], ws_ref[...])


def _tc_pre(x, nc, wn, ws):
    d = x.shape[1]
    return pl.pallas_call(
        _pre_body,
        grid=(_GRID,),
        in_specs=[_rows_spec(d), _rows_spec(1), _full_spec(d, H), _full_spec(d, H)],
        out_specs=[_rows_spec(WSC), _rows_spec(H)],
        out_shape=[jax.ShapeDtypeStruct((N, WSC), f32),
                   jax.ShapeDtypeStruct((N, H), f32)],
    )(x, nc, wn, ws)


def _agg(p0_ref, p1_ref):
    q = p0_ref[...] + p1_ref[...]
    return lax.slice(q, (0, 0), (_RB, H))


def _combine_body(use_nc, *refs):
    if use_nc:
        p0_ref, p1_ref, s_ref, b_ref, nc_ref, wn_ref, ws_ref, y_ref, sn_ref = refs
    else:
        p0_ref, p1_ref, s_ref, b_ref, wn_ref, ws_ref, y_ref, sn_ref = refs
    h = _pelu(_agg(p0_ref, p1_ref) + s_ref[...] + b_ref[...])
    y_ref[...] = _widen(_dot(h, wn_ref[...]))
    hs = h * nc_ref[...] if use_nc else h
    sn_ref[...] = _dot(hs, ws_ref[...])


def _tc_combine(p, s, b, nc, wn, ws, use_nc):
    in_specs = [_rows_spec(WSC), _rows_spec(WSC), _rows_spec(H), _full_spec(1, H)]
    args = [p[0], p[1], s, b]
    if use_nc:
        in_specs.append(_rows_spec(1))
        args.append(nc)
    in_specs += [_full_spec(H, H), _full_spec(H, H)]
    args += [wn, ws]
    return pl.pallas_call(
        functools.partial(_combine_body, use_nc),
        grid=(_GRID,),
        in_specs=in_specs,
        out_specs=[_rows_spec(WSC), _rows_spec(H)],
        out_shape=[jax.ShapeDtypeStruct((N, WSC), f32),
                   jax.ShapeDtypeStruct((N, H), f32)],
    )(*args)


def _finish1_body(p0_ref, p1_ref, s_ref, b_ref, batch_ref, hext_ref, x1_ref):
    i = pl.program_id(0)
    h = _pelu(_agg(p0_ref, p1_ref) + s_ref[...] + b_ref[...])
    ones = jnp.ones((_RB, 1), f32)
    zeros = jnp.zeros((_RB, WSC - H - 1), f32)
    hext_ref[...] = jnp.concatenate([h, ones, zeros], axis=1)
    onehot = (batch_ref[...] == lax.broadcasted_iota(i32, (1, B), 1)).astype(f32)
    x1b = _dot_t(onehot, h)

    @pl.when(i == 0)
    def _():
        x1_ref[...] = jnp.zeros_like(x1_ref)

    x1_ref[...] += x1b


def _tc_finish1(p, s, b, batch):
    return pl.pallas_call(
        _finish1_body,
        grid=(_GRID,),
        in_specs=[_rows_spec(WSC), _rows_spec(WSC), _rows_spec(H),
                  _full_spec(1, H), _rows_spec(1)],
        out_specs=[_rows_spec(WSC), _full_spec(B, H)],
        out_shape=[jax.ShapeDtypeStruct((N, WSC), f32),
                   jax.ShapeDtypeStruct((B, H), f32)],
    )(p[0], p[1], s, b, batch)


def _poolhead_body(q0_ref, q1_ref, iso_ref, wna_ref, wnb_ref, wsa_ref, wsb_ref,
                   y_ref, s_ref):
    q = q0_ref[...] + q1_ref[...]
    cnt = jnp.maximum(lax.slice(q, (0, H), (_RB, H + 1)), 1.0)
    pool = lax.slice(q, (0, 0), (_RB, H)) / cnt
    iso = iso_ref[...]
    y_ref[...] = _widen(_dot(pool, wna_ref[...]) + _dot(iso, wnb_ref[...]))
    s_ref[...] = _dot(pool, wsa_ref[...]) + _dot(iso, wsb_ref[...])


def _tc_poolhead(q0, q1, iso, wn, ws):
    ni = iso.shape[1]
    return pl.pallas_call(
        _poolhead_body,
        grid=(_GRID,),
        in_specs=[_rows_spec(WSC), _rows_spec(WSC), _rows_spec(ni),
                  _full_spec(H, H), _full_spec(ni, H),
                  _full_spec(H, H), _full_spec(ni, H)],
        out_specs=[_rows_spec(WSC), _rows_spec(H)],
        out_shape=[jax.ShapeDtypeStruct((N, WSC), f32),
                   jax.ShapeDtypeStruct((N, H), f32)],
    )(q0, q1, iso, wn[:H], wn[H:], ws[:H], ws[H:])


def _finish2_body(p0_ref, p1_ref, s_ref, b_ref, batch_ref, xs_ref, cnt_ref):
    i = pl.program_id(0)
    h = _pelu(_agg(p0_ref, p1_ref) + s_ref[...] + b_ref[...])
    onehot = (batch_ref[...] == lax.broadcasted_iota(i32, (1, B), 1)).astype(f32)
    xsb = _dot_t(onehot, h)
    cntb = _dot_t(onehot, jnp.ones((_RB, 1), f32))

    @pl.when(i == 0)
    def _():
        xs_ref[...] = jnp.zeros_like(xs_ref)
        cnt_ref[...] = jnp.zeros_like(cnt_ref)

    xs_ref[...] += xsb
    cnt_ref[...] += cntb


def _tc_finish2(p, s, b, batch):
    return pl.pallas_call(
        _finish2_body,
        grid=(_GRID,),
        in_specs=[_rows_spec(WSC), _rows_spec(WSC), _rows_spec(H),
                  _full_spec(1, H), _rows_spec(1)],
        out_specs=[_full_spec(B, H), _full_spec(B, 1)],
        out_shape=[jax.ShapeDtypeStruct((B, H), f32),
                   jax.ShapeDtypeStruct((B, 1), f32)],
    )(p[0], p[1], s, b, batch)


def _head_body(x1_ref, x2s_ref, c2_ref, x3s_ref, c3_ref,
               w1a_ref, w1b_ref, w1c_ref, b1_ref, w2_ref, b2_ref,
               w3_ref, b3_ref, o_ref):
    x1 = x1_ref[...]
    x2 = x2s_ref[...] / jnp.maximum(c2_ref[...], 1.0)
    x3 = x3s_ref[...] / jnp.maximum(c3_ref[...], 1.0)
    z = (_dot(x1, w1a_ref[...]) + _dot(x2, w1b_ref[...])
         + _dot(x3, w1c_ref[...]) + b1_ref[...])
    z = _pelu(z)
    z = _pelu(_dot(z, w2_ref[...]) + b2_ref[...])
    z = _dot(z, w3_ref[...]) + b3_ref[...]
    m = jnp.max(z, axis=1, keepdims=True)
    lse = jnp.log(jnp.sum(jnp.exp(z - m), axis=1, keepdims=True)) + m
    o_ref[...] = z - lse


def _tc_head(x1, x2s, c2, x3s, c3, p):
    hh = H // 2
    return pl.pallas_call(
        _head_body,
        grid=(1,),
        in_specs=[_full_spec(B, H), _full_spec(B, H), _full_spec(B, 1),
                  _full_spec(B, H), _full_spec(B, 1),
                  _full_spec(H, H), _full_spec(H, H), _full_spec(H, H),
                  _full_spec(1, H), _full_spec(H, hh), _full_spec(1, hh),
                  _full_spec(hh, NCLS), _full_spec(1, NCLS)],
        out_specs=pl.BlockSpec((B, NCLS), lambda i: (0, 0)),
        out_shape=jax.ShapeDtypeStruct((B, NCLS), f32),
    )(x1, x2s, c2, x3s, c3, p['Wf1'][:H], p['Wf1'][H:2 * H], p['Wf1'][2 * H:],
      p['bf1'][None, :], p['Wf2'], p['bf2'][None, :], p['Wf3'], p['bf3'][None, :])


def _pad_assign(assign, total):
    pad = total - assign.shape[1]
    src = jnp.concatenate([assign[0], jnp.zeros((pad,), i32)])
    dst = jnp.concatenate([assign[1], jnp.full((pad,), N, i32)])
    return src, dst


def kernel(x, edge_index, node_centrality1, edge_centrality1, batch,
           assignment_index_2, iso_type_2, edge_index_2, batch_2,
           assignment_index_3, iso_type_3, edge_index_3, batch_3, params):
    p = params
    nc1 = node_centrality1[:, None]
    src1, dst1 = edge_index[0], edge_index[1]
    ec1 = edge_centrality1

    y, s = _tc_pre(x, nc1, p['W1n'], p['W1s'])
    agg = _sc_conv(y, src1, dst1, ec1, with_ec=True)
    y, s = _tc_combine(agg, s, p['b1'][None, :], nc1, p['W2n'], p['W2s'], True)
    agg = _sc_conv(y, src1, dst1, ec1, with_ec=True)
    y, s = _tc_combine(agg, s, p['b2'][None, :], nc1, p['W3n'], p['W3s'], True)
    agg = _sc_conv(y, src1, dst1, ec1, with_ec=True)
    hext, x1 = _tc_finish1(agg, s, p['b3'][None, :], batch[:, None])

    s2a, d2a = _pad_assign(assignment_index_2, 40960)
    q = _sc_pool(hext, s2a, d2a)
    y, s = _tc_poolhead(q[0, :N], q[1, :N], iso_type_2, p['W4n'], p['W4s'])
    agg = _sc_conv(y, edge_index_2[0], edge_index_2[1], ec1, with_ec=False)
    y, s = _tc_combine(agg, s, p['b4'][None, :], None, p['W5n'], p['W5s'], False)
    agg = _sc_conv(y, edge_index_2[0], edge_index_2[1], ec1, with_ec=False)
    x2s, c2 = _tc_finish2(agg, s, p['b5'][None, :], batch_2[:, None])

    s3a, d3a = _pad_assign(assignment_index_3, 61440)
    q = _sc_pool(hext, s3a, d3a)
    y, s = _tc_poolhead(q[0, :N], q[1, :N], iso_type_3, p['W6n'], p['W6s'])
    agg = _sc_conv(y, edge_index_3[0], edge_index_3[1], ec1, with_ec=False)
    y, s = _tc_combine(agg, s, p['b6'][None, :], None, p['W7n'], p['W7s'], False)
    agg = _sc_conv(y, edge_index_3[0], edge_index_3[1], ec1, with_ec=False)
    x3s, c3 = _tc_finish2(agg, s, p['b7'][None, :], batch_3[:, None])

    return _tc_head(x1, x2s, c2, x3s, c3, p)


# speedup vs baseline: 5.2455x; 1.5671x over previous
import dataclasses
import functools

import jax
import jax.numpy as jnp
from jax import lax
from jax.experimental import pallas as pl
from jax.experimental.pallas import tpu as pltpu
from jax.experimental.pallas import tpu_sc as plsc

N = 10000
H = 64
B = 64
NCLS = 10
NC_SC = 2
NS_SC = 16
NW = NC_SC * NS_SC
CH = 80
WSC = 128
ROWS_PER_TILE = N // NS_SC
NPOOL = 10016
PROWS_PER_TILE = NPOOL // NS_SC

f32 = jnp.float32
i32 = jnp.int32


def _pelu(z):
    return jnp.where(z > 0, z, jnp.exp(jnp.minimum(z, 0.0)) - 1.0)


def _sc_mesh():
    return plsc.VectorSubcoreMesh(
        core_axis_name="c", subcore_axis_name="s",
        num_cores=NC_SC, num_subcores=NS_SC)


def _sc_params():
    cp = pltpu.CompilerParams()
    if "needs_layout_passes" in pltpu.CompilerParams.__dataclass_fields__:
        cp = dataclasses.replace(cp, needs_layout_passes=False)
    return cp


def _zero_fill(buf, nrows):
    zv = jnp.zeros((16,), f32)

    @pl.loop(0, nrows)
    def _(r):
        for j in range(WSC // 16):
            buf[r, pl.ds(j * 16, 16)] = zv


def _zero_acc(zbuf, acc_sh, base, nrows, sem):
    _zero_fill(zbuf, CH)
    full, rem = divmod(nrows, CH)
    descs = [pltpu.make_async_copy(
        zbuf, acc_sh.at[pl.ds(base + k * CH, CH)], sem) for k in range(full)]
    if rem:
        descs.append(pltpu.make_async_copy(
            zbuf.at[pl.ds(0, rem)],
            acc_sh.at[pl.ds(base + full * CH, rem)], sem))
    for d in descs:
        d.start()
    for d in descs:
        d.wait()


def _stream_body(with_ec, nch, rpt, tab_hbm, pidx3_hbm, ec4_hbm,
                 out_hbm, pidx_all, sbuf0, sbuf1, dbuf0, dbuf1,
                 ecb0, ecb1, rows0, rows1,
                 acc_sh, sem_z, sem_g0, sem_g1, sem_s0, sem_s1, sem_e0, sem_e1):
    cid = lax.axis_index("c")
    sid = lax.axis_index("s")
    wid = cid * NS_SC + sid
    rowsb = (rows0, rows1)
    sbuf = (sbuf0, sbuf1)
    dbuf = (dbuf0, dbuf1)
    ecb = (ecb0, ecb1)
    sem_g = (sem_g0, sem_g1)
    sem_s = (sem_s0, sem_s1)
    sem_e = (sem_e0, sem_e1)

    _zero_acc(rows0, acc_sh, sid * rpt, rpt, sem_z)
    pltpu.sync_copy(pidx3_hbm.at[wid], pidx_all)
    plsc.subcore_barrier()

    def unpack(c, b):
        for j in range(CH // 16):
            sl = pl.ds(j * 16, 16)
            v = pidx_all[c, sl]
            sbuf[b][sl] = v & 0xFFFF
            dbuf[b][sl] = lax.shift_right_logical(v, 16)

    def g_desc(c, b):
        return pltpu.make_async_copy(tab_hbm.at[sbuf[b]], rowsb[b], sem_g[b])

    def e_desc(c, b):
        return pltpu.make_async_copy(ec4_hbm.at[wid, c], ecb[b], sem_e[b])

    def g_start(c, b):
        g_desc(c, b).start()
        if with_ec:
            e_desc(c, b).start()

    def g_wait(c, b):
        g_desc(c, b).wait()
        if with_ec:
            e_desc(c, b).wait()

    def s_desc(c, b):
        return pltpu.make_async_copy(rowsb[b], acc_sh.at[dbuf[b]], sem_s[b])

    def mult(b):
        if not with_ec:
            return

        @pl.loop(0, CH)
        def _(e):
            ecv = plsc.load_gather(
                ecb[b], [jnp.zeros((16,), i32), jnp.full((16,), e, i32)])
            for j in range(H // 16):
                sl = pl.ds(j * 16, 16)
                rowsb[b][e, sl] = rowsb[b][e, sl] * ecv

    unpack(0, 0)
    g_start(0, 0)
    g_wait(0, 0)
    mult(0)
    s_desc(0, 0).start(add=True)
    unpack(1, 1)
    g_start(1, 1)

    @pl.loop(0, (nch - 1) // 2)
    def _(p):
        for k in range(2):
            c = 2 * p + 1 + k
            b = 1 - k
            g_wait(c, b)
            mult(b)
            s_desc(c, b).start(add=True)
            s_desc(c - 1, 1 - b).wait()

            @pl.when(c + 1 < nch)
            def _():
                unpack(c + 1, 1 - b)
                g_start(c + 1, 1 - b)

    s_desc(nch - 1, (nch - 1) % 2).wait()
    plsc.subcore_barrier()
    sl = pl.ds(sid * rpt, rpt)
    pltpu.sync_copy(acc_sh.at[sl], out_hbm.at[cid, sid])


def _sc_scatter_add(tab, src, dst, ec, nacc, with_ec):
    e = src.shape[0]
    nch = e // (NW * CH)
    assert nch % 2 == 1 and nch * NW * CH == e
    rpt = nacc // NS_SC
    pidx3 = (src + dst * 65536).reshape(NW, nch, CH)
    ec4 = ec.reshape(NW, nch, 1, CH) if with_ec else jnp.zeros((NW, 1, 1, CH), f32)
    scratch = [
        pltpu.VMEM((nch, CH), i32),
        pltpu.VMEM((CH,), i32),
        pltpu.VMEM((CH,), i32),
        pltpu.VMEM((CH,), i32),
        pltpu.VMEM((CH,), i32),
        pltpu.VMEM((1, CH), f32),
        pltpu.VMEM((1, CH), f32),
        pltpu.VMEM((CH, WSC), f32),
        pltpu.VMEM((CH, WSC), f32),
        pltpu.VMEM_SHARED((nacc, WSC), f32),
    ] + [pltpu.SemaphoreType.DMA] * 7
    kern = pl.kernel(
        functools.partial(_stream_body, with_ec, nch, rpt),
        out_type=jax.ShapeDtypeStruct((NC_SC, NS_SC, rpt, WSC), f32),
        mesh=_sc_mesh(),
        scratch_types=scratch,
        compiler_params=_sc_params(),
    )
    return kern(tab, pidx3, ec4).reshape(NC_SC, nacc, WSC)


def _sc_conv(y, src, dst, ec, *, with_ec):
    return _sc_scatter_add(y, src, dst, ec, N, with_ec)


def _sc_pool(tab, src, dst):
    return _sc_scatter_add(tab, src, dst, tab, NPOOL, False)


_RB = 1000
_GRID = N // _RB


def _dot(a, b):
    return lax.dot_general(a, b, (((1,), (0,)), ((), ())),
                           preferred_element_type=f32,
                           precision=lax.Precision.HIGHEST)


def _dot_t(a, b):
    return lax.dot_general(a, b, (((0,), (0,)), ((), ())),
                           preferred_element_type=f32,
                           precision=lax.Precision.HIGHEST)


def _rows_spec(w):
    return pl.BlockSpec((_RB, w), lambda i: (i, 0))


def _full_spec(r, c):
    return pl.BlockSpec((r, c), lambda i: (0, 0))


def _widen(y):
    return jnp.concatenate([y, jnp.zeros((_RB, WSC - H), f32)], axis=1)


def _pre_body(x_ref, nc_ref, wn_ref, ws_ref, y_ref, s_ref):
    xb = x_ref[...]
    y_ref[...] = _widen(_dot(xb, wn_ref[...]))
    s_ref[...] = _dot(xb * nc_ref[...], ws_ref[...])


def _tc_pre(x, nc, wn, ws):
    d = x.shape[1]
    return pl.pallas_call(
        _pre_body,
        grid=(_GRID,),
        in_specs=[_rows_spec(d), _rows_spec(1), _full_spec(d, H), _full_spec(d, H)],
        out_specs=[_rows_spec(WSC), _rows_spec(H)],
        out_shape=[jax.ShapeDtypeStruct((N, WSC), f32),
                   jax.ShapeDtypeStruct((N, H), f32)],
    )(x, nc, wn, ws)


def _agg(p0_ref, p1_ref):
    q = p0_ref[...] + p1_ref[...]
    return lax.slice(q, (0, 0), (_RB, H))


def _combine_body(use_nc, *refs):
    if use_nc:
        p0_ref, p1_ref, s_ref, b_ref, nc_ref, wn_ref, ws_ref, y_ref, sn_ref = refs
    else:
        p0_ref, p1_ref, s_ref, b_ref, wn_ref, ws_ref, y_ref, sn_ref = refs
    h = _pelu(_agg(p0_ref, p1_ref) + s_ref[...] + b_ref[...])
    y_ref[...] = _widen(_dot(h, wn_ref[...]))
    hs = h * nc_ref[...] if use_nc else h
    sn_ref[...] = _dot(hs, ws_ref[...])


def _tc_combine(p, s, b, nc, wn, ws, use_nc):
    in_specs = [_rows_spec(WSC), _rows_spec(WSC), _rows_spec(H), _full_spec(1, H)]
    args = [p[0], p[1], s, b]
    if use_nc:
        in_specs.append(_rows_spec(1))
        args.append(nc)
    in_specs += [_full_spec(H, H), _full_spec(H, H)]
    args += [wn, ws]
    return pl.pallas_call(
        functools.partial(_combine_body, use_nc),
        grid=(_GRID,),
        in_specs=in_specs,
        out_specs=[_rows_spec(WSC), _rows_spec(H)],
        out_shape=[jax.ShapeDtypeStruct((N, WSC), f32),
                   jax.ShapeDtypeStruct((N, H), f32)],
    )(*args)


def _finish1_body(p0_ref, p1_ref, s_ref, b_ref, batch_ref, hext_ref, x1_ref):
    i = pl.program_id(0)
    h = _pelu(_agg(p0_ref, p1_ref) + s_ref[...] + b_ref[...])
    ones = jnp.ones((_RB, 1), f32)
    zeros = jnp.zeros((_RB, WSC - H - 1), f32)
    hext_ref[...] = jnp.concatenate([h, ones, zeros], axis=1)
    onehot = (batch_ref[...] == lax.broadcasted_iota(i32, (1, B), 1)).astype(f32)
    x1b = _dot_t(onehot, h)

    @pl.when(i == 0)
    def _():
        x1_ref[...] = jnp.zeros_like(x1_ref)

    x1_ref[...] += x1b


def _tc_finish1(p, s, b, batch):
    return pl.pallas_call(
        _finish1_body,
        grid=(_GRID,),
        in_specs=[_rows_spec(WSC), _rows_spec(WSC), _rows_spec(H),
                  _full_spec(1, H), _rows_spec(1)],
        out_specs=[_rows_spec(WSC), _full_spec(B, H)],
        out_shape=[jax.ShapeDtypeStruct((N, WSC), f32),
                   jax.ShapeDtypeStruct((B, H), f32)],
    )(p[0], p[1], s, b, batch)


def _poolhead_body(q0_ref, q1_ref, iso_ref, wna_ref, wnb_ref, wsa_ref, wsb_ref,
                   y_ref, s_ref):
    q = q0_ref[...] + q1_ref[...]
    cnt = jnp.maximum(lax.slice(q, (0, H), (_RB, H + 1)), 1.0)
    pool = lax.slice(q, (0, 0), (_RB, H)) / cnt
    iso = iso_ref[...]
    y_ref[...] = _widen(_dot(pool, wna_ref[...]) + _dot(iso, wnb_ref[...]))
    s_ref[...] = _dot(pool, wsa_ref[...]) + _dot(iso, wsb_ref[...])


def _tc_poolhead(q0, q1, iso, wn, ws):
    ni = iso.shape[1]
    return pl.pallas_call(
        _poolhead_body,
        grid=(_GRID,),
        in_specs=[_rows_spec(WSC), _rows_spec(WSC), _rows_spec(ni),
                  _full_spec(H, H), _full_spec(ni, H),
                  _full_spec(H, H), _full_spec(ni, H)],
        out_specs=[_rows_spec(WSC), _rows_spec(H)],
        out_shape=[jax.ShapeDtypeStruct((N, WSC), f32),
                   jax.ShapeDtypeStruct((N, H), f32)],
    )(q0, q1, iso, wn[:H], wn[H:], ws[:H], ws[H:])


def _finish2_body(p0_ref, p1_ref, s_ref, b_ref, batch_ref, xs_ref, cnt_ref):
    i = pl.program_id(0)
    h = _pelu(_agg(p0_ref, p1_ref) + s_ref[...] + b_ref[...])
    onehot = (batch_ref[...] == lax.broadcasted_iota(i32, (1, B), 1)).astype(f32)
    xsb = _dot_t(onehot, h)
    cntb = _dot_t(onehot, jnp.ones((_RB, 1), f32))

    @pl.when(i == 0)
    def _():
        xs_ref[...] = jnp.zeros_like(xs_ref)
        cnt_ref[...] = jnp.zeros_like(cnt_ref)

    xs_ref[...] += xsb
    cnt_ref[...] += cntb


def _tc_finish2(p, s, b, batch):
    return pl.pallas_call(
        _finish2_body,
        grid=(_GRID,),
        in_specs=[_rows_spec(WSC), _rows_spec(WSC), _rows_spec(H),
                  _full_spec(1, H), _rows_spec(1)],
        out_specs=[_full_spec(B, H), _full_spec(B, 1)],
        out_shape=[jax.ShapeDtypeStruct((B, H), f32),
                   jax.ShapeDtypeStruct((B, 1), f32)],
    )(p[0], p[1], s, b, batch)


def _head_body(x1_ref, x2s_ref, c2_ref, x3s_ref, c3_ref,
               w1a_ref, w1b_ref, w1c_ref, b1_ref, w2_ref, b2_ref,
               w3_ref, b3_ref, o_ref):
    x1 = x1_ref[...]
    x2 = x2s_ref[...] / jnp.maximum(c2_ref[...], 1.0)
    x3 = x3s_ref[...] / jnp.maximum(c3_ref[...], 1.0)
    z = (_dot(x1, w1a_ref[...]) + _dot(x2, w1b_ref[...])
         + _dot(x3, w1c_ref[...]) + b1_ref[...])
    z = _pelu(z)
    z = _pelu(_dot(z, w2_ref[...]) + b2_ref[...])
    z = _dot(z, w3_ref[...]) + b3_ref[...]
    m = jnp.max(z, axis=1, keepdims=True)
    lse = jnp.log(jnp.sum(jnp.exp(z - m), axis=1, keepdims=True)) + m
    o_ref[...] = z - lse


def _tc_head(x1, x2s, c2, x3s, c3, p):
    hh = H // 2
    return pl.pallas_call(
        _head_body,
        grid=(1,),
        in_specs=[_full_spec(B, H), _full_spec(B, H), _full_spec(B, 1),
                  _full_spec(B, H), _full_spec(B, 1),
                  _full_spec(H, H), _full_spec(H, H), _full_spec(H, H),
                  _full_spec(1, H), _full_spec(H, hh), _full_spec(1, hh),
                  _full_spec(hh, NCLS), _full_spec(1, NCLS)],
        out_specs=pl.BlockSpec((B, NCLS), lambda i: (0, 0)),
        out_shape=jax.ShapeDtypeStruct((B, NCLS), f32),
    )(x1, x2s, c2, x3s, c3, p['Wf1'][:H], p['Wf1'][H:2 * H], p['Wf1'][2 * H:],
      p['bf1'][None, :], p['Wf2'], p['bf2'][None, :], p['Wf3'], p['bf3'][None, :])


def _pad_assign(assign, total):
    pad = total - assign.shape[1]
    src = jnp.concatenate([assign[0], jnp.zeros((pad,), i32)])
    dst = jnp.concatenate([assign[1], jnp.full((pad,), N, i32)])
    return src, dst


def kernel(x, edge_index, node_centrality1, edge_centrality1, batch,
           assignment_index_2, iso_type_2, edge_index_2, batch_2,
           assignment_index_3, iso_type_3, edge_index_3, batch_3, params):
    p = params
    nc1 = node_centrality1[:, None]
    src1, dst1 = edge_index[0], edge_index[1]
    ec1 = edge_centrality1

    y, s = _tc_pre(x, nc1, p['W1n'], p['W1s'])
    agg = _sc_conv(y, src1, dst1, ec1, with_ec=True)
    y, s = _tc_combine(agg, s, p['b1'][None, :], nc1, p['W2n'], p['W2s'], True)
    agg = _sc_conv(y, src1, dst1, ec1, with_ec=True)
    y, s = _tc_combine(agg, s, p['b2'][None, :], nc1, p['W3n'], p['W3s'], True)
    agg = _sc_conv(y, src1, dst1, ec1, with_ec=True)
    hext, x1 = _tc_finish1(agg, s, p['b3'][None, :], batch[:, None])

    s2a, d2a = _pad_assign(assignment_index_2, 43520)
    q = _sc_pool(hext, s2a, d2a)
    y, s = _tc_poolhead(q[0, :N], q[1, :N], iso_type_2, p['W4n'], p['W4s'])
    agg = _sc_conv(y, edge_index_2[0], edge_index_2[1], ec1, with_ec=False)
    y, s = _tc_combine(agg, s, p['b4'][None, :], None, p['W5n'], p['W5s'], False)
    agg = _sc_conv(y, edge_index_2[0], edge_index_2[1], ec1, with_ec=False)
    x2s, c2 = _tc_finish2(agg, s, p['b5'][None, :], batch_2[:, None])

    s3a, d3a = _pad_assign(assignment_index_3, 64000)
    q = _sc_pool(hext, s3a, d3a)
    y, s = _tc_poolhead(q[0, :N], q[1, :N], iso_type_3, p['W6n'], p['W6s'])
    agg = _sc_conv(y, edge_index_3[0], edge_index_3[1], ec1, with_ec=False)
    y, s = _tc_combine(agg, s, p['b6'][None, :], None, p['W7n'], p['W7s'], False)
    agg = _sc_conv(y, edge_index_3[0], edge_index_3[1], ec1, with_ec=False)
    x3s, c3 = _tc_finish2(agg, s, p['b7'][None, :], batch_3[:, None])

    return _tc_head(x1, x2s, c2, x3s, c3, p)


# speedup vs baseline: 6.3699x; 1.2144x over previous
import dataclasses
import functools

import jax
import jax.numpy as jnp
from jax import lax
from jax.experimental import pallas as pl
from jax.experimental.pallas import tpu as pltpu
from jax.experimental.pallas import tpu_sc as plsc

N = 10000
H = 64
B = 64
NCLS = 10
NC_SC = 2
NS_SC = 16
NW = NC_SC * NS_SC
CH = 80
PW = 80
NPOOL = 10016

f32 = jnp.float32
i32 = jnp.int32


def _pelu(z):
    return jnp.where(z > 0, z, jnp.exp(jnp.minimum(z, 0.0)) - 1.0)


def _sc_mesh():
    return plsc.VectorSubcoreMesh(
        core_axis_name="c", subcore_axis_name="s",
        num_cores=NC_SC, num_subcores=NS_SC)


def _sc_params():
    cp = pltpu.CompilerParams()
    fields = pltpu.CompilerParams.__dataclass_fields__
    if "needs_layout_passes" in fields:
        cp = dataclasses.replace(cp, needs_layout_passes=False)
    if "use_tc_tiling_on_sc" in fields:
        cp = dataclasses.replace(cp, use_tc_tiling_on_sc=False)
    return cp


def _zero_fill(buf, nrows, w):
    zv = jnp.zeros((16,), f32)

    @pl.loop(0, nrows, unroll=4)
    def _(r):
        for j in range(w // 16):
            buf[r, pl.ds(j * 16, 16)] = zv


def _zero_acc(zbuf, acc_sh, base, nrows, sem):
    _zero_fill(zbuf, CH, zbuf.shape[1])
    full, rem = divmod(nrows, CH)
    descs = [pltpu.make_async_copy(
        zbuf, acc_sh.at[pl.ds(base + k * CH, CH)], sem) for k in range(full)]
    if rem:
        descs.append(pltpu.make_async_copy(
            zbuf.at[pl.ds(0, rem)],
            acc_sh.at[pl.ds(base + full * CH, rem)], sem))
    for d in descs:
        d.start()
    for d in descs:
        d.wait()


def _stream_body(with_ec, nch, rpt, w, tab_hbm, pidx3_hbm, ec4_hbm,
                 out_hbm, pidx_all, sbuf0, sbuf1, dbuf0, dbuf1,
                 ecb0, ecb1, rows0, rows1,
                 acc_sh, sem_z, sem_g0, sem_g1, sem_s0, sem_s1, sem_e0, sem_e1):
    cid = lax.axis_index("c")
    sid = lax.axis_index("s")
    wid = cid * NS_SC + sid
    rowsb = (rows0, rows1)
    sbuf = (sbuf0, sbuf1)
    dbuf = (dbuf0, dbuf1)
    ecb = (ecb0, ecb1)
    sem_g = (sem_g0, sem_g1)
    sem_s = (sem_s0, sem_s1)
    sem_e = (sem_e0, sem_e1)

    _zero_acc(rows0, acc_sh, sid * rpt, rpt, sem_z)
    pltpu.sync_copy(pidx3_hbm.at[wid], pidx_all)
    plsc.subcore_barrier()

    def unpack(c, b):
        for j in range(CH // 16):
            sl = pl.ds(j * 16, 16)
            v = pidx_all[c, sl]
            sbuf[b][sl] = v & 0xFFFF
            dbuf[b][sl] = lax.shift_right_logical(v, 16)

    def g_desc(c, b):
        return pltpu.make_async_copy(tab_hbm.at[sbuf[b]], rowsb[b], sem_g[b])

    def e_desc(c, b):
        return pltpu.make_async_copy(ec4_hbm.at[wid, c], ecb[b], sem_e[b])

    def g_start(c, b):
        g_desc(c, b).start()
        if with_ec:
            e_desc(c, b).start()

    def g_wait(c, b):
        g_desc(c, b).wait()
        if with_ec:
            e_desc(c, b).wait()

    def s_desc(c, b):
        return pltpu.make_async_copy(rowsb[b], acc_sh.at[dbuf[b]], sem_s[b])

    def mult(b):
        if not with_ec:
            return

        @pl.loop(0, CH, unroll=2)
        def _(e):
            ecv = plsc.load_gather(
                ecb[b], [jnp.zeros((16,), i32), jnp.full((16,), e, i32)])
            for j in range(H // 16):
                sl = pl.ds(j * 16, 16)
                rowsb[b][e, sl] = rowsb[b][e, sl] * ecv

    unpack(0, 0)
    g_start(0, 0)
    g_wait(0, 0)
    mult(0)
    s_desc(0, 0).start(add=True)
    unpack(1, 1)
    g_start(1, 1)

    @pl.loop(0, (nch - 1) // 2)
    def _(p):
        for k in range(2):
            c = 2 * p + 1 + k
            b = 1 - k
            g_wait(c, b)
            mult(b)
            s_desc(c, b).start(add=True)
            s_desc(c - 1, 1 - b).wait()

            @pl.when(c + 1 < nch)
            def _():
                unpack(c + 1, 1 - b)
                g_start(c + 1, 1 - b)

    s_desc(nch - 1, (nch - 1) % 2).wait()
    plsc.subcore_barrier()
    sl = pl.ds(sid * rpt, rpt)
    pltpu.sync_copy(acc_sh.at[sl], out_hbm.at[cid, sid])


def _sc_scatter_add(tab, src, dst, ec, nacc, with_ec):
    e = src.shape[0]
    w = tab.shape[1]
    nch = e // (NW * CH)
    assert nch % 2 == 1 and nch * NW * CH == e
    rpt = nacc // NS_SC
    pidx3 = (src + dst * 65536).reshape(NW, nch, CH)
    ec4 = ec.reshape(NW, nch, 1, CH) if with_ec else jnp.zeros((NW, 1, 1, CH), f32)
    scratch = [
        pltpu.VMEM((nch, CH), i32),
        pltpu.VMEM((CH,), i32),
        pltpu.VMEM((CH,), i32),
        pltpu.VMEM((CH,), i32),
        pltpu.VMEM((CH,), i32),
        pltpu.VMEM((1, CH), f32),
        pltpu.VMEM((1, CH), f32),
        pltpu.VMEM((CH, w), f32),
        pltpu.VMEM((CH, w), f32),
        pltpu.VMEM_SHARED((nacc, w), f32),
    ] + [pltpu.SemaphoreType.DMA] * 7
    kern = pl.kernel(
        functools.partial(_stream_body, with_ec, nch, rpt, w),
        out_type=jax.ShapeDtypeStruct((NC_SC, NS_SC, rpt, w), f32),
        mesh=_sc_mesh(),
        scratch_types=scratch,
        compiler_params=_sc_params(),
    )
    return kern(tab, pidx3, ec4).reshape(NC_SC, nacc, w)


def _sc_conv(y, src, dst, ec, *, with_ec):
    return _sc_scatter_add(y, src, dst, ec, N, with_ec)


def _sc_pool(tab, src, dst):
    return _sc_scatter_add(tab, src, dst, tab, NPOOL, False)


_RB = 1000
_GRID = N // _RB


def _dot(a, b):
    return lax.dot_general(a, b, (((1,), (0,)), ((), ())),
                           preferred_element_type=f32,
                           precision=lax.Precision.HIGHEST)


def _dot_t(a, b):
    return lax.dot_general(a, b, (((0,), (0,)), ((), ())),
                           preferred_element_type=f32,
                           precision=lax.Precision.HIGHEST)


def _rows_spec(w):
    return pl.BlockSpec((_RB, w), lambda i: (i, 0))


def _full_spec(r, c):
    return pl.BlockSpec((r, c), lambda i: (0, 0))


def _pre_body(x_ref, nc_ref, wn_ref, ws_ref, y_ref, s_ref):
    xb = x_ref[...]
    y_ref[...] = _dot(xb, wn_ref[...])
    s_ref[...] = _dot(xb * nc_ref[...], ws_ref[...])


def _tc_pre(x, nc, wn, ws):
    d = x.shape[1]
    return pl.pallas_call(
        _pre_body,
        grid=(_GRID,),
        in_specs=[_rows_spec(d), _rows_spec(1), _full_spec(d, H), _full_spec(d, H)],
        out_specs=[_rows_spec(H), _rows_spec(H)],
        out_shape=[jax.ShapeDtypeStruct((N, H), f32)] * 2,
    )(x, nc, wn, ws)


def _combine_body(use_nc, *refs):
    if use_nc:
        p0_ref, p1_ref, s_ref, b_ref, nc_ref, wn_ref, ws_ref, y_ref, sn_ref = refs
    else:
        p0_ref, p1_ref, s_ref, b_ref, wn_ref, ws_ref, y_ref, sn_ref = refs
    h = _pelu(p0_ref[...] + p1_ref[...] + s_ref[...] + b_ref[...])
    y_ref[...] = _dot(h, wn_ref[...])
    hs = h * nc_ref[...] if use_nc else h
    sn_ref[...] = _dot(hs, ws_ref[...])


def _tc_combine(p, s, b, nc, wn, ws, use_nc):
    in_specs = [_rows_spec(H), _rows_spec(H), _rows_spec(H), _full_spec(1, H)]
    args = [p[0], p[1], s, b]
    if use_nc:
        in_specs.append(_rows_spec(1))
        args.append(nc)
    in_specs += [_full_spec(H, H), _full_spec(H, H)]
    args += [wn, ws]
    return pl.pallas_call(
        functools.partial(_combine_body, use_nc),
        grid=(_GRID,),
        in_specs=in_specs,
        out_specs=[_rows_spec(H), _rows_spec(H)],
        out_shape=[jax.ShapeDtypeStruct((N, H), f32)] * 2,
    )(*args)


def _finish1_body(p0_ref, p1_ref, s_ref, b_ref, batch_ref, hext_ref, x1_ref):
    i = pl.program_id(0)
    h = _pelu(p0_ref[...] + p1_ref[...] + s_ref[...] + b_ref[...])
    ones = jnp.ones((_RB, 1), f32)
    zeros = jnp.zeros((_RB, PW - H - 1), f32)
    hext_ref[...] = jnp.concatenate([h, ones, zeros], axis=1)
    onehot = (batch_ref[...] == lax.broadcasted_iota(i32, (1, B), 1)).astype(f32)
    x1b = _dot_t(onehot, h)

    @pl.when(i == 0)
    def _():
        x1_ref[...] = jnp.zeros_like(x1_ref)

    x1_ref[...] += x1b


def _tc_finish1(p, s, b, batch):
    return pl.pallas_call(
        _finish1_body,
        grid=(_GRID,),
        in_specs=[_rows_spec(H), _rows_spec(H), _rows_spec(H),
                  _full_spec(1, H), _rows_spec(1)],
        out_specs=[_rows_spec(PW), _full_spec(B, H)],
        out_shape=[jax.ShapeDtypeStruct((N, PW), f32),
                   jax.ShapeDtypeStruct((B, H), f32)],
    )(p[0], p[1], s, b, batch)


def _poolhead_body(q0_ref, q1_ref, iso_ref, wna_ref, wnb_ref, wsa_ref, wsb_ref,
                   y_ref, s_ref):
    q = q0_ref[...] + q1_ref[...]
    cnt = jnp.maximum(lax.slice(q, (0, H), (_RB, H + 1)), 1.0)
    pool = lax.slice(q, (0, 0), (_RB, H)) / cnt
    iso = iso_ref[...]
    y_ref[...] = _dot(pool, wna_ref[...]) + _dot(iso, wnb_ref[...])
    s_ref[...] = _dot(pool, wsa_ref[...]) + _dot(iso, wsb_ref[...])


def _tc_poolhead(q0, q1, iso, wn, ws):
    ni = iso.shape[1]
    return pl.pallas_call(
        _poolhead_body,
        grid=(_GRID,),
        in_specs=[_rows_spec(PW), _rows_spec(PW), _rows_spec(ni),
                  _full_spec(H, H), _full_spec(ni, H),
                  _full_spec(H, H), _full_spec(ni, H)],
        out_specs=[_rows_spec(H), _rows_spec(H)],
        out_shape=[jax.ShapeDtypeStruct((N, H), f32)] * 2,
    )(q0, q1, iso, wn[:H], wn[H:], ws[:H], ws[H:])


def _finish2_body(p0_ref, p1_ref, s_ref, b_ref, batch_ref, xs_ref, cnt_ref):
    i = pl.program_id(0)
    h = _pelu(p0_ref[...] + p1_ref[...] + s_ref[...] + b_ref[...])
    onehot = (batch_ref[...] == lax.broadcasted_iota(i32, (1, B), 1)).astype(f32)
    xsb = _dot_t(onehot, h)
    cntb = _dot_t(onehot, jnp.ones((_RB, 1), f32))

    @pl.when(i == 0)
    def _():
        xs_ref[...] = jnp.zeros_like(xs_ref)
        cnt_ref[...] = jnp.zeros_like(cnt_ref)

    xs_ref[...] += xsb
    cnt_ref[...] += cntb


def _tc_finish2(p, s, b, batch):
    return pl.pallas_call(
        _finish2_body,
        grid=(_GRID,),
        in_specs=[_rows_spec(H), _rows_spec(H), _rows_spec(H),
                  _full_spec(1, H), _rows_spec(1)],
        out_specs=[_full_spec(B, H), _full_spec(B, 1)],
        out_shape=[jax.ShapeDtypeStruct((B, H), f32),
                   jax.ShapeDtypeStruct((B, 1), f32)],
    )(p[0], p[1], s, b, batch)


def _head_body(x1_ref, x2s_ref, c2_ref, x3s_ref, c3_ref,
               w1a_ref, w1b_ref, w1c_ref, b1_ref, w2_ref, b2_ref,
               w3_ref, b3_ref, o_ref):
    x1 = x1_ref[...]
    x2 = x2s_ref[...] / jnp.maximum(c2_ref[...], 1.0)
    x3 = x3s_ref[...] / jnp.maximum(c3_ref[...], 1.0)
    z = (_dot(x1, w1a_ref[...]) + _dot(x2, w1b_ref[...])
         + _dot(x3, w1c_ref[...]) + b1_ref[...])
    z = _pelu(z)
    z = _pelu(_dot(z, w2_ref[...]) + b2_ref[...])
    z = _dot(z, w3_ref[...]) + b3_ref[...]
    m = jnp.max(z, axis=1, keepdims=True)
    lse = jnp.log(jnp.sum(jnp.exp(z - m), axis=1, keepdims=True)) + m
    o_ref[...] = z - lse


def _tc_head(x1, x2s, c2, x3s, c3, p):
    hh = H // 2
    return pl.pallas_call(
        _head_body,
        grid=(1,),
        in_specs=[_full_spec(B, H), _full_spec(B, H), _full_spec(B, 1),
                  _full_spec(B, H), _full_spec(B, 1),
                  _full_spec(H, H), _full_spec(H, H), _full_spec(H, H),
                  _full_spec(1, H), _full_spec(H, hh), _full_spec(1, hh),
                  _full_spec(hh, NCLS), _full_spec(1, NCLS)],
        out_specs=pl.BlockSpec((B, NCLS), lambda i: (0, 0)),
        out_shape=jax.ShapeDtypeStruct((B, NCLS), f32),
    )(x1, x2s, c2, x3s, c3, p['Wf1'][:H], p['Wf1'][H:2 * H], p['Wf1'][2 * H:],
      p['bf1'][None, :], p['Wf2'], p['bf2'][None, :], p['Wf3'], p['bf3'][None, :])


def _pad_assign(assign, total):
    pad = total - assign.shape[1]
    src = jnp.concatenate([assign[0], jnp.zeros((pad,), i32)])
    dst = jnp.concatenate([assign[1], jnp.full((pad,), N, i32)])
    return src, dst


def kernel(x, edge_index, node_centrality1, edge_centrality1, batch,
           assignment_index_2, iso_type_2, edge_index_2, batch_2,
           assignment_index_3, iso_type_3, edge_index_3, batch_3, params):
    p = params
    nc1 = node_centrality1[:, None]
    src1, dst1 = edge_index[0], edge_index[1]
    ec1 = edge_centrality1

    y, s = _tc_pre(x, nc1, p['W1n'], p['W1s'])
    agg = _sc_conv(y, src1, dst1, ec1, with_ec=True)
    y, s = _tc_combine(agg, s, p['b1'][None, :], nc1, p['W2n'], p['W2s'], True)
    agg = _sc_conv(y, src1, dst1, ec1, with_ec=True)
    y, s = _tc_combine(agg, s, p['b2'][None, :], nc1, p['W3n'], p['W3s'], True)
    agg = _sc_conv(y, src1, dst1, ec1, with_ec=True)
    hext, x1 = _tc_finish1(agg, s, p['b3'][None, :], batch[:, None])

    s2a, d2a = _pad_assign(assignment_index_2, 43520)
    q = _sc_pool(hext, s2a, d2a)
    y, s = _tc_poolhead(q[0, :N], q[1, :N], iso_type_2, p['W4n'], p['W4s'])
    agg = _sc_conv(y, edge_index_2[0], edge_index_2[1], ec1, with_ec=False)
    y, s = _tc_combine(agg, s, p['b4'][None, :], None, p['W5n'], p['W5s'], False)
    agg = _sc_conv(y, edge_index_2[0], edge_index_2[1], ec1, with_ec=False)
    x2s, c2 = _tc_finish2(agg, s, p['b5'][None, :], batch_2[:, None])

    s3a, d3a = _pad_assign(assignment_index_3, 64000)
    q = _sc_pool(hext, s3a, d3a)
    y, s = _tc_poolhead(q[0, :N], q[1, :N], iso_type_3, p['W6n'], p['W6s'])
    agg = _sc_conv(y, edge_index_3[0], edge_index_3[1], ec1, with_ec=False)
    y, s = _tc_combine(agg, s, p['b6'][None, :], None, p['W7n'], p['W7s'], False)
    agg = _sc_conv(y, edge_index_3[0], edge_index_3[1], ec1, with_ec=False)
    x3s, c3 = _tc_finish2(agg, s, p['b7'][None, :], batch_3[:, None])

    return _tc_head(x1, x2s, c2, x3s, c3, p)
